# baseline (speedup 1.0000x reference)
import jax
import jax.numpy as jnp
from jax.experimental import pallas as pl

_NEG = -1e9
_TINY = 1.1754943508222875e-38
_KS0 = 0x0
_KS1 = 0x2A
_KS2 = _KS0 ^ _KS1 ^ 0x1BD11BDA
_ROT_A = (13, 15, 26, 6)
_ROT_B = (17, 29, 16, 24)


def _rotl(x, d):
    return (x << jnp.uint32(d)) | (x >> jnp.uint32(32 - d))


def _threefry_rounds(x0, x1, rots):
    for r in rots:
        x0 = x0 + x1
        x1 = _rotl(x1, r)
        x1 = x0 ^ x1
    return x0, x1


def _threefry_bits(cnt_hi, cnt_lo):
    ks0 = jnp.uint32(_KS0)
    ks1 = jnp.uint32(_KS1)
    ks2 = jnp.uint32(_KS2)
    x0 = cnt_hi + ks0
    x1 = cnt_lo + ks1
    x0, x1 = _threefry_rounds(x0, x1, _ROT_A)
    x0, x1 = x0 + ks1, x1 + ks2 + jnp.uint32(1)
    x0, x1 = _threefry_rounds(x0, x1, _ROT_B)
    x0, x1 = x0 + ks2, x1 + ks0 + jnp.uint32(2)
    x0, x1 = _threefry_rounds(x0, x1, _ROT_A)
    x0, x1 = x0 + ks0, x1 + ks1 + jnp.uint32(3)
    x0, x1 = _threefry_rounds(x0, x1, _ROT_B)
    x0, x1 = x0 + ks1, x1 + ks2 + jnp.uint32(4)
    x0, x1 = _threefry_rounds(x0, x1, _ROT_A)
    x0, x1 = x0 + ks2, x1 + ks0 + jnp.uint32(5)
    return x0 ^ x1


def _body(logits_ref, mask_ref, out_ref, *, vshift):
    blk_r, v = logits_ref.shape
    logits = logits_ref[...]
    mask = mask_ref[...]
    masked = jnp.where(mask, logits, jnp.float32(_NEG))

    m = jnp.max(masked, axis=1, keepdims=True)
    s = jnp.sum(jnp.exp(masked - m), axis=1, keepdims=True)
    lse = m + jnp.log(s)

    row0 = jnp.uint32(pl.program_id(0) * blk_r)
    rowi = jax.lax.broadcasted_iota(jnp.uint32, (blk_r, v), 0)
    coli = jax.lax.broadcasted_iota(jnp.uint32, (blk_r, v), 1)
    cnt_lo = ((row0 + rowi) << jnp.uint32(vshift)) | coli
    bits = _threefry_bits(jnp.zeros((blk_r, v), jnp.uint32), cnt_lo)

    fb = (bits >> jnp.uint32(9)) | jnp.uint32(0x3F800000)
    f = jax.lax.bitcast_convert_type(fb, jnp.float32) - jnp.float32(1.0)
    tiny = jnp.float32(_TINY)
    u = jnp.maximum(tiny, f * (jnp.float32(1.0) - tiny) + tiny)
    g = -jnp.log(-jnp.log(u))

    y = g + masked
    ymax = jnp.max(y, axis=1, keepdims=True)
    ci32 = coli.astype(jnp.int32)
    amax = jnp.min(jnp.where(y == ymax, ci32, jnp.int32(v)), axis=1,
                   keepdims=True)
    val = jnp.sum(jnp.where(ci32 == amax, masked, jnp.float32(0.0)), axis=1,
                  keepdims=True)
    out_ref[...] = val - lse


def kernel(logits, mask):
    b, v = logits.shape
    assert (v & (v - 1)) == 0, "V must be a power of two"
    vshift = v.bit_length() - 1
    blk_r = 512 if b % 512 == 0 else b

    import functools
    out = pl.pallas_call(
        functools.partial(_body, vshift=vshift),
        grid=(b // blk_r,),
        in_specs=[
            pl.BlockSpec((blk_r, v), lambda i: (i, 0)),
            pl.BlockSpec((blk_r, v), lambda i: (i, 0)),
        ],
        out_specs=pl.BlockSpec((blk_r, 1), lambda i: (i, 0)),
        out_shape=jax.ShapeDtypeStruct((b, 1), jnp.float32),
    )(logits, mask)
    return out.reshape(b)


# speedup vs baseline: 1.2488x; 1.2488x over previous
import functools

import jax
import jax.numpy as jnp
from jax.experimental import pallas as pl

_NEG = -1e9
_TINY = 1.1754943508222875e-38
_KS0 = 0x0
_KS1 = 0x2A
_KS2 = _KS0 ^ _KS1 ^ 0x1BD11BDA
_ROT_A = (13, 15, 26, 6)
_ROT_B = (17, 29, 16, 24)


def _rotl(x, d):
    return (x << jnp.uint32(d)) | (x >> jnp.uint32(32 - d))


def _threefry_rounds(x0, x1, rots):
    for r in rots:
        x0 = x0 + x1
        x1 = _rotl(x1, r)
        x1 = x0 ^ x1
    return x0, x1


def _threefry_bits(cnt_lo):
    ks0 = jnp.uint32(_KS0)
    ks1 = jnp.uint32(_KS1)
    ks2 = jnp.uint32(_KS2)
    x1i = cnt_lo + ks1
    x0 = x1i
    x1 = x0 ^ _rotl(x1i, _ROT_A[0])
    x0, x1 = _threefry_rounds(x0, x1, _ROT_A[1:])
    x0, x1 = x0 + ks1, x1 + ks2 + jnp.uint32(1)
    x0, x1 = _threefry_rounds(x0, x1, _ROT_B)
    x0, x1 = x0 + ks2, x1 + ks0 + jnp.uint32(2)
    x0, x1 = _threefry_rounds(x0, x1, _ROT_A)
    x0, x1 = x0 + ks0, x1 + ks1 + jnp.uint32(3)
    x0, x1 = _threefry_rounds(x0, x1, _ROT_B)
    x0, x1 = x0 + ks1, x1 + ks2 + jnp.uint32(4)
    x0, x1 = _threefry_rounds(x0, x1, _ROT_A)
    x0, x1 = x0 + ks2, x1 + ks0 + jnp.uint32(5)
    return x0 ^ x1


def _body(iota_ref, logits_ref, mask_ref, out_ref, *, vshift):
    blk_r, v = logits_ref.shape
    logits = logits_ref[...]
    mask = mask_ref[...]
    masked = jnp.where(mask, logits, jnp.float32(_NEG))

    m = jnp.max(masked, axis=1, keepdims=True)
    s = jnp.sum(jnp.exp(masked - m), axis=1, keepdims=True)
    lse = m + jnp.log(s)

    base = jnp.uint32(pl.program_id(0)) * jnp.uint32(blk_r << vshift)
    cnt = iota_ref[...] + base
    bits = _threefry_bits(cnt)

    fb = (bits >> jnp.uint32(9)) | jnp.uint32(0x3F800000)
    f = jax.lax.bitcast_convert_type(fb, jnp.float32) - jnp.float32(1.0)
    u = f + jnp.float32(_TINY)
    g = -jnp.log(-jnp.log(u))

    y = g + masked
    ymax = jnp.max(y, axis=1, keepdims=True)
    val = jnp.max(jnp.where(y == ymax, masked, jnp.float32(-jnp.inf)),
                  axis=1, keepdims=True)
    out_ref[...] = val - lse


def kernel(logits, mask):
    b, v = logits.shape
    assert (v & (v - 1)) == 0, "V must be a power of two"
    vshift = v.bit_length() - 1
    blk_r = 256 if b % 256 == 0 else b

    rowi = jax.lax.broadcasted_iota(jnp.uint32, (blk_r, v), 0)
    coli = jax.lax.broadcasted_iota(jnp.uint32, (blk_r, v), 1)
    blk_iota = (rowi << vshift) | coli

    out = pl.pallas_call(
        functools.partial(_body, vshift=vshift),
        grid=(b // blk_r,),
        in_specs=[
            pl.BlockSpec((blk_r, v), lambda i: (0, 0)),
            pl.BlockSpec((blk_r, v), lambda i: (i, 0)),
            pl.BlockSpec((blk_r, v), lambda i: (i, 0)),
        ],
        out_specs=pl.BlockSpec((blk_r, 1), lambda i: (i, 0)),
        out_shape=jax.ShapeDtypeStruct((b, 1), jnp.float32),
    )(blk_iota, logits, mask)
    return out.reshape(b)


# speedup vs baseline: 1.3338x; 1.0681x over previous
import functools

import jax
import jax.numpy as jnp
from jax import lax
from jax.experimental import pallas as pl
from jax.experimental.pallas import tpu as pltpu
from jax.experimental.pallas import tpu_sc as plsc

_NEG = -1e9
_TINY = 1.1754943508222875e-38
_KS0 = 0x0
_KS1 = 0x2A
_KS2 = _KS0 ^ _KS1 ^ 0x1BD11BDA
_ROT_A = (13, 15, 26, 6)
_ROT_B = (17, 29, 16, 24)


def _rotl(x, d):
    return (x << jnp.uint32(d)) | (x >> jnp.uint32(32 - d))


def _threefry_rounds(x0, x1, rots):
    for r in rots:
        x0 = x0 + x1
        x1 = _rotl(x1, r)
        x1 = x0 ^ x1
    return x0, x1


def _threefry_bits(cnt_lo):
    ks0 = jnp.uint32(_KS0)
    ks1 = jnp.uint32(_KS1)
    ks2 = jnp.uint32(_KS2)
    x1i = cnt_lo + ks1
    x0 = x1i
    x1 = x0 ^ _rotl(x1i, _ROT_A[0])
    x0, x1 = _threefry_rounds(x0, x1, _ROT_A[1:])
    x0, x1 = x0 + ks1, x1 + ks2 + jnp.uint32(1)
    x0, x1 = _threefry_rounds(x0, x1, _ROT_B)
    x0, x1 = x0 + ks2, x1 + ks0 + jnp.uint32(2)
    x0, x1 = _threefry_rounds(x0, x1, _ROT_A)
    x0, x1 = x0 + ks0, x1 + ks1 + jnp.uint32(3)
    x0, x1 = _threefry_rounds(x0, x1, _ROT_B)
    x0, x1 = x0 + ks1, x1 + ks2 + jnp.uint32(4)
    x0, x1 = _threefry_rounds(x0, x1, _ROT_A)
    x0, x1 = x0 + ks2, x1 + ks0 + jnp.uint32(5)
    return x0 ^ x1


def _body(iota_ref, logits_ref, mask_ref, out_ref, *, vshift, blk_off):
    blk_r, v = logits_ref.shape
    logits = logits_ref[...]
    mask = mask_ref[...]
    masked = jnp.where(mask, logits, jnp.float32(_NEG))

    m = jnp.max(masked, axis=1, keepdims=True)
    s = jnp.sum(jnp.exp(masked - m), axis=1, keepdims=True)
    lse = m + jnp.log(s)

    base = (jnp.uint32(pl.program_id(0)) + jnp.uint32(blk_off)) * jnp.uint32(
        blk_r << vshift)
    cnt = iota_ref[...] + base
    bits = _threefry_bits(cnt)

    fb = (bits >> jnp.uint32(9)) | jnp.uint32(0x3F800000)
    f = jax.lax.bitcast_convert_type(fb, jnp.float32) - jnp.float32(1.0)
    u = f + jnp.float32(_TINY)
    g = -jnp.log(-jnp.log(u))

    y = g + masked
    ymax = jnp.max(y, axis=1, keepdims=True)
    val = jnp.max(jnp.where(y == ymax, masked, jnp.float32(-jnp.inf)),
                  axis=1, keepdims=True)
    out_ref[...] = val - lse


_SC_W = 32
_SC_RPW = 48
_SC_ROWS = _SC_W * _SC_RPW
_LN2 = 0.6931471805599453
_SQRT2 = 1.4142135623730951


def _sc_ln(x):
    bits = lax.bitcast_convert_type(x, jnp.int32)
    eb = (bits >> 23) & jnp.int32(0xFF)
    mb = (bits & jnp.int32(0x7FFFFF)) | jnp.int32(0x3F800000)
    m = lax.bitcast_convert_type(mb, jnp.float32)
    big = m >= jnp.float32(_SQRT2)
    m = jnp.where(big, m * jnp.float32(0.5), m)
    eb = jnp.where(big, eb + jnp.int32(1), eb)
    ef = lax.bitcast_convert_type(eb + jnp.int32(0x4B000000),
                                  jnp.float32) - jnp.float32(8388608.0 + 127.0)
    d = m + jnp.float32(1.0)
    q = lax.bitcast_convert_type(
        jnp.int32(0x7EF127EA) - lax.bitcast_convert_type(d, jnp.int32),
        jnp.float32)
    q = q * (jnp.float32(2.0) - d * q)
    q = q * (jnp.float32(2.0) - d * q)
    q = q * (jnp.float32(2.0) - d * q)
    r = (m - jnp.float32(1.0)) * q
    r2 = r * r
    p = jnp.float32(1.0 / 9.0)
    p = p * r2 + jnp.float32(1.0 / 7.0)
    p = p * r2 + jnp.float32(1.0 / 5.0)
    p = p * r2 + jnp.float32(1.0 / 3.0)
    p = p * r2 + jnp.float32(1.0)
    return ef * jnp.float32(_LN2) + (r + r) * p


def _sc_allreduce(x, lane, op):
    for k in (8, 4, 2, 1):
        y = x.at[lane ^ jnp.int32(k)].get(mode="promise_in_bounds")
        x = op(x, y)
    return x


def _sc_body(logits_hbm, maski_hbm, out_hbm, lbuf, mbuf, obuf):
    wid = lax.axis_index("s") * jnp.int32(2) + lax.axis_index("c")
    base = wid * jnp.int32(_SC_RPW)
    lane = lax.iota(jnp.int32, 16)
    lane_u = lax.bitcast_convert_type(lane, jnp.uint32)
    neg = jnp.full((16,), _NEG, jnp.float32)
    ninf = jnp.full((16,), -jnp.inf, jnp.float32)
    zf = jnp.zeros((16,), jnp.float32)
    zi = jnp.zeros((16,), jnp.int32)

    def row_step(r, res_vec):
        row = base + r
        pltpu.sync_copy(logits_hbm.at[row], lbuf)
        pltpu.sync_copy(maski_hbm.at[row], mbuf)

        def p1(c, m_acc):
            sl = pl.ds(c * 16, 16)
            msk = jnp.where(mbuf[sl] != 0, lbuf[sl], neg)
            lbuf[sl] = msk
            return jnp.maximum(m_acc, msk)

        m_acc = lax.fori_loop(0, 256, p1, neg)
        msplat = _sc_allreduce(m_acc, lane, jnp.maximum)
        cbase = lax.convert_element_type(row << 12, jnp.uint32)

        def p2(c, carry):
            s_acc, ym, il, vl = carry
            msk = lbuf[pl.ds(c * 16, 16)]
            s_acc = s_acc + jnp.exp(msk - msplat)
            cnt = jnp.full((16,), cbase + lax.convert_element_type(
                c * 16, jnp.uint32), jnp.uint32) + lane_u
            bits = _threefry_bits(cnt)
            fb = (bits >> jnp.uint32(9)) | jnp.uint32(0x3F800000)
            f = lax.bitcast_convert_type(fb, jnp.float32) - jnp.float32(1.0)
            u = f + jnp.float32(_TINY)
            g = -_sc_ln(-_sc_ln(u))
            y = g + msk
            upd = y > ym
            ym = jnp.where(upd, y, ym)
            il = jnp.where(upd, jnp.full((16,), c, jnp.int32), il)
            vl = jnp.where(upd, msk, vl)
            return s_acc, ym, il, vl

        s_acc, ym, il, vl = lax.fori_loop(0, 256, p2, (zf, ninf, zi, neg))
        srow = _sc_allreduce(s_acc, lane, jnp.add)
        lse = msplat + _sc_ln(srow)
        ymax = _sc_allreduce(ym, lane, jnp.maximum)
        eidx = il * jnp.int32(16) + lane
        cand = ym == ymax
        widx = _sc_allreduce(jnp.where(cand, eidx, jnp.int32(1 << 30)),
                             lane, jnp.minimum)
        val = _sc_allreduce(jnp.where(cand & (eidx == widx), vl, ninf),
                            lane, jnp.maximum)
        res = val - lse
        res_vec = jnp.where(lane == (r & 15), res, res_vec)
        obuf[pl.ds((r // 16) * 16, 16)] = res_vec
        return res_vec

    lax.fori_loop(0, _SC_RPW, row_step, zf)
    pltpu.sync_copy(obuf, out_hbm.at[pl.ds(base, _SC_RPW)])


def _sc_sample(logits, maski):
    mesh = plsc.VectorSubcoreMesh(core_axis_name="c", subcore_axis_name="s")
    return pl.kernel(
        _sc_body,
        mesh=mesh,
        out_type=jax.ShapeDtypeStruct((_SC_ROWS,), jnp.float32),
        scratch_types=[
            pltpu.VMEM((4096,), jnp.float32),
            pltpu.VMEM((4096,), jnp.int32),
            pltpu.VMEM((_SC_RPW,), jnp.float32),
        ],
    )(logits, maski)


def kernel(logits, mask):
    b, v = logits.shape
    assert (v & (v - 1)) == 0, "V must be a power of two"
    vshift = v.bit_length() - 1
    blk_r = 256 if b % 256 == 0 else b

    use_sc = v == 4096 and b % 256 == 0 and b > 2 * _SC_ROWS
    sc_rows = _SC_ROWS if use_sc else 0
    tc_rows = b - sc_rows
    blk_off = sc_rows // blk_r

    rowi = jax.lax.broadcasted_iota(jnp.uint32, (blk_r, v), 0)
    coli = jax.lax.broadcasted_iota(jnp.uint32, (blk_r, v), 1)
    blk_iota = (rowi << vshift) | coli

    out_tc = pl.pallas_call(
        functools.partial(_body, vshift=vshift, blk_off=blk_off),
        grid=(tc_rows // blk_r,),
        in_specs=[
            pl.BlockSpec((blk_r, v), lambda i: (0, 0)),
            pl.BlockSpec((blk_r, v), lambda i: (i + blk_off, 0)),
            pl.BlockSpec((blk_r, v), lambda i: (i + blk_off, 0)),
        ],
        out_specs=pl.BlockSpec((blk_r, 1), lambda i: (i, 0)),
        out_shape=jax.ShapeDtypeStruct((tc_rows, 1), jnp.float32),
    )(blk_iota, logits, mask)
    out_tc = out_tc.reshape(tc_rows)
    if not use_sc:
        return out_tc
    out_sc = _sc_sample(logits, mask[:_SC_ROWS].astype(jnp.int32))
    return jnp.concatenate([out_sc, out_tc])


# speedup vs baseline: 1.3762x; 1.0317x over previous
import functools

import jax
import jax.numpy as jnp
from jax import lax
from jax.experimental import pallas as pl
from jax.experimental.pallas import tpu as pltpu
from jax.experimental.pallas import tpu_sc as plsc

_NEG = -1e9
_TINY = 1.1754943508222875e-38
_KS0 = 0x0
_KS1 = 0x2A
_KS2 = _KS0 ^ _KS1 ^ 0x1BD11BDA
_ROT_A = (13, 15, 26, 6)
_ROT_B = (17, 29, 16, 24)


def _rotl(x, d):
    return (x << jnp.uint32(d)) | (x >> jnp.uint32(32 - d))


def _threefry_rounds(x0, x1, rots):
    for r in rots:
        x0 = x0 + x1
        x1 = _rotl(x1, r)
        x1 = x0 ^ x1
    return x0, x1


def _threefry_bits(cnt_lo):
    ks0 = jnp.uint32(_KS0)
    ks1 = jnp.uint32(_KS1)
    ks2 = jnp.uint32(_KS2)
    x1i = cnt_lo + ks1
    x0 = x1i
    x1 = x0 ^ _rotl(x1i, _ROT_A[0])
    x0, x1 = _threefry_rounds(x0, x1, _ROT_A[1:])
    x0, x1 = x0 + ks1, x1 + ks2 + jnp.uint32(1)
    x0, x1 = _threefry_rounds(x0, x1, _ROT_B)
    x0, x1 = x0 + ks2, x1 + ks0 + jnp.uint32(2)
    x0, x1 = _threefry_rounds(x0, x1, _ROT_A)
    x0, x1 = x0 + ks0, x1 + ks1 + jnp.uint32(3)
    x0, x1 = _threefry_rounds(x0, x1, _ROT_B)
    x0, x1 = x0 + ks1, x1 + ks2 + jnp.uint32(4)
    x0, x1 = _threefry_rounds(x0, x1, _ROT_A)
    x0, x1 = x0 + ks2, x1 + ks0 + jnp.uint32(5)
    return x0 ^ x1


def _body(iota_ref, logits_ref, mask_ref, out_ref, *, vshift, blk_off):
    blk_r, v = logits_ref.shape
    logits = logits_ref[...]
    mask = mask_ref[...]
    masked = jnp.where(mask, logits, jnp.float32(_NEG))

    m = jnp.max(masked, axis=1, keepdims=True)
    s = jnp.sum(jnp.exp(masked - m), axis=1, keepdims=True)
    lse = m + jnp.log(s)

    base = (jnp.uint32(pl.program_id(0)) + jnp.uint32(blk_off)) * jnp.uint32(
        blk_r << vshift)
    cnt = iota_ref[...] + base
    bits = _threefry_bits(cnt)

    fb = (bits >> jnp.uint32(9)) | jnp.uint32(0x3F800000)
    f = jax.lax.bitcast_convert_type(fb, jnp.float32) - jnp.float32(1.0)
    u = f + jnp.float32(_TINY)
    g = -jnp.log(-jnp.log(u))

    y = g + masked
    ymax = jnp.max(y, axis=1, keepdims=True)
    val = jnp.max(jnp.where(y == ymax, masked, jnp.float32(-jnp.inf)),
                  axis=1, keepdims=True)
    out_ref[...] = val - lse


_SC_W = 32
_SC_RPW = 64
_SC_ROWS = _SC_W * _SC_RPW
_LN2 = 0.6931471805599453
_SQRT2 = 1.4142135623730951


def _sc_ln(x):
    bits = lax.bitcast_convert_type(x, jnp.int32)
    eb = (bits >> 23) & jnp.int32(0xFF)
    mb = (bits & jnp.int32(0x7FFFFF)) | jnp.int32(0x3F800000)
    m = lax.bitcast_convert_type(mb, jnp.float32)
    big = m >= jnp.float32(_SQRT2)
    m = jnp.where(big, m * jnp.float32(0.5), m)
    eb = jnp.where(big, eb + jnp.int32(1), eb)
    ef = lax.bitcast_convert_type(eb + jnp.int32(0x4B000000),
                                  jnp.float32) - jnp.float32(8388608.0 + 127.0)
    d = m + jnp.float32(1.0)
    q = lax.bitcast_convert_type(
        jnp.int32(0x7EF127EA) - lax.bitcast_convert_type(d, jnp.int32),
        jnp.float32)
    q = q * (jnp.float32(2.0) - d * q)
    q = q * (jnp.float32(2.0) - d * q)
    q = q * (jnp.float32(2.0) - d * q)
    r = (m - jnp.float32(1.0)) * q
    r2 = r * r
    p = jnp.float32(1.0 / 9.0)
    p = p * r2 + jnp.float32(1.0 / 7.0)
    p = p * r2 + jnp.float32(1.0 / 5.0)
    p = p * r2 + jnp.float32(1.0 / 3.0)
    p = p * r2 + jnp.float32(1.0)
    return ef * jnp.float32(_LN2) + (r + r) * p


def _sc_allreduce(x, lane, op):
    for k in (8, 4, 2, 1):
        y = x.at[lane ^ jnp.int32(k)].get(mode="promise_in_bounds")
        x = op(x, y)
    return x


def _sc_body(logits_hbm, maski_hbm, out_hbm, lbuf, mbuf, obuf):
    wid = lax.axis_index("s") * jnp.int32(2) + lax.axis_index("c")
    base = wid * jnp.int32(_SC_RPW)
    lane = lax.iota(jnp.int32, 16)
    lane_u = lax.bitcast_convert_type(lane, jnp.uint32)
    neg = jnp.full((16,), _NEG, jnp.float32)
    ninf = jnp.full((16,), -jnp.inf, jnp.float32)
    zf = jnp.zeros((16,), jnp.float32)
    zi = jnp.zeros((16,), jnp.int32)

    def row_step(r, res_vec):
        row = base + r
        pltpu.sync_copy(logits_hbm.at[row], lbuf)
        pltpu.sync_copy(maski_hbm.at[row], mbuf)

        def p1(c, m_acc):
            sl = pl.ds(c * 16, 16)
            msk = jnp.where(mbuf[sl] != 0, lbuf[sl], neg)
            lbuf[sl] = msk
            return jnp.maximum(m_acc, msk)

        m_acc = lax.fori_loop(0, 256, p1, neg)
        msplat = _sc_allreduce(m_acc, lane, jnp.maximum)
        cbase = lax.convert_element_type(row << 12, jnp.uint32)

        def p2(c, carry):
            s_acc, ym, il, vl = carry
            msk = lbuf[pl.ds(c * 16, 16)]
            s_acc = s_acc + jnp.exp(msk - msplat)
            cnt = jnp.full((16,), cbase + lax.convert_element_type(
                c * 16, jnp.uint32), jnp.uint32) + lane_u
            bits = _threefry_bits(cnt)
            fb = (bits >> jnp.uint32(9)) | jnp.uint32(0x3F800000)
            f = lax.bitcast_convert_type(fb, jnp.float32) - jnp.float32(1.0)
            u = f + jnp.float32(_TINY)
            g = -_sc_ln(-_sc_ln(u))
            y = g + msk
            upd = y > ym
            ym = jnp.where(upd, y, ym)
            il = jnp.where(upd, jnp.full((16,), c, jnp.int32), il)
            vl = jnp.where(upd, msk, vl)
            return s_acc, ym, il, vl

        s_acc, ym, il, vl = lax.fori_loop(0, 256, p2, (zf, ninf, zi, neg))
        srow = _sc_allreduce(s_acc, lane, jnp.add)
        lse = msplat + _sc_ln(srow)
        ymax = _sc_allreduce(ym, lane, jnp.maximum)
        eidx = il * jnp.int32(16) + lane
        cand = ym == ymax
        widx = _sc_allreduce(jnp.where(cand, eidx, jnp.int32(1 << 30)),
                             lane, jnp.minimum)
        val = _sc_allreduce(jnp.where(cand & (eidx == widx), vl, ninf),
                            lane, jnp.maximum)
        res = val - lse
        res_vec = jnp.where(lane == (r & 15), res, res_vec)
        obuf[pl.ds((r // 16) * 16, 16)] = res_vec
        return res_vec

    lax.fori_loop(0, _SC_RPW, row_step, zf)
    pltpu.sync_copy(obuf, out_hbm.at[pl.ds(base, _SC_RPW)])


def _sc_sample(logits, maski):
    mesh = plsc.VectorSubcoreMesh(core_axis_name="c", subcore_axis_name="s")
    return pl.kernel(
        _sc_body,
        mesh=mesh,
        out_type=jax.ShapeDtypeStruct((_SC_ROWS,), jnp.float32),
        scratch_types=[
            pltpu.VMEM((4096,), jnp.float32),
            pltpu.VMEM((4096,), jnp.int32),
            pltpu.VMEM((_SC_RPW,), jnp.float32),
        ],
    )(logits, maski)


def kernel(logits, mask):
    b, v = logits.shape
    assert (v & (v - 1)) == 0, "V must be a power of two"
    vshift = v.bit_length() - 1
    blk_r = 256 if b % 256 == 0 else b

    use_sc = v == 4096 and b % 256 == 0 and b > 2 * _SC_ROWS
    sc_rows = _SC_ROWS if use_sc else 0
    tc_rows = b - sc_rows
    blk_off = sc_rows // blk_r

    rowi = jax.lax.broadcasted_iota(jnp.uint32, (blk_r, v), 0)
    coli = jax.lax.broadcasted_iota(jnp.uint32, (blk_r, v), 1)
    blk_iota = (rowi << vshift) | coli

    out_tc = pl.pallas_call(
        functools.partial(_body, vshift=vshift, blk_off=blk_off),
        grid=(tc_rows // blk_r,),
        in_specs=[
            pl.BlockSpec((blk_r, v), lambda i: (0, 0)),
            pl.BlockSpec((blk_r, v), lambda i: (i + blk_off, 0)),
            pl.BlockSpec((blk_r, v), lambda i: (i + blk_off, 0)),
        ],
        out_specs=pl.BlockSpec((blk_r, 1), lambda i: (i, 0)),
        out_shape=jax.ShapeDtypeStruct((tc_rows, 1), jnp.float32),
    )(blk_iota, logits, mask)
    out_tc = out_tc.reshape(tc_rows)
    if not use_sc:
        return out_tc
    out_sc = _sc_sample(logits, mask[:_SC_ROWS].astype(jnp.int32))
    return jnp.concatenate([out_sc, out_tc])


# speedup vs baseline: 1.4164x; 1.0293x over previous
import functools

import jax
import jax.numpy as jnp
from jax import lax
from jax.experimental import pallas as pl
from jax.experimental.pallas import tpu as pltpu
from jax.experimental.pallas import tpu_sc as plsc

_NEG = -1e9
_TINY = 1.1754943508222875e-38
_KS0 = 0x0
_KS1 = 0x2A
_KS2 = _KS0 ^ _KS1 ^ 0x1BD11BDA
_ROT_A = (13, 15, 26, 6)
_ROT_B = (17, 29, 16, 24)


def _rotl(x, d):
    return (x << jnp.uint32(d)) | (x >> jnp.uint32(32 - d))


def _threefry_rounds(x0, x1, rots):
    for r in rots:
        x0 = x0 + x1
        x1 = _rotl(x1, r)
        x1 = x0 ^ x1
    return x0, x1


def _threefry_bits(cnt_lo):
    ks0 = jnp.uint32(_KS0)
    ks1 = jnp.uint32(_KS1)
    ks2 = jnp.uint32(_KS2)
    x1i = cnt_lo + ks1
    x0 = x1i
    x1 = x0 ^ _rotl(x1i, _ROT_A[0])
    x0, x1 = _threefry_rounds(x0, x1, _ROT_A[1:])
    x0, x1 = x0 + ks1, x1 + ks2 + jnp.uint32(1)
    x0, x1 = _threefry_rounds(x0, x1, _ROT_B)
    x0, x1 = x0 + ks2, x1 + ks0 + jnp.uint32(2)
    x0, x1 = _threefry_rounds(x0, x1, _ROT_A)
    x0, x1 = x0 + ks0, x1 + ks1 + jnp.uint32(3)
    x0, x1 = _threefry_rounds(x0, x1, _ROT_B)
    x0, x1 = x0 + ks1, x1 + ks2 + jnp.uint32(4)
    x0, x1 = _threefry_rounds(x0, x1, _ROT_A)
    x0, x1 = x0 + ks2, x1 + ks0 + jnp.uint32(5)
    return x0 ^ x1


def _body(iota_ref, logits_ref, mask_ref, out_ref, *, vshift, blk_off):
    blk_r, v = logits_ref.shape
    logits = logits_ref[...]
    mask = mask_ref[...]
    masked = jnp.where(mask, logits, jnp.float32(_NEG))

    m = jnp.max(masked, axis=1, keepdims=True)
    s = jnp.sum(jnp.exp(masked - m), axis=1, keepdims=True)
    lse = m + jnp.log(s)

    base = (jnp.uint32(pl.program_id(0)) + jnp.uint32(blk_off)) * jnp.uint32(
        blk_r << vshift)
    cnt = iota_ref[...] + base
    bits = _threefry_bits(cnt)

    fb = (bits >> jnp.uint32(9)) | jnp.uint32(0x3F800000)
    f = jax.lax.bitcast_convert_type(fb, jnp.float32) - jnp.float32(1.0)
    u = f + jnp.float32(_TINY)
    g = -jnp.log(-jnp.log(u))

    y = g + masked
    ymax = jnp.max(y, axis=1, keepdims=True)
    val = jnp.max(jnp.where(y == ymax, masked, jnp.float32(-jnp.inf)),
                  axis=1, keepdims=True)
    out_ref[...] = val - lse


_SC_W = 32
_SC_RPW = 80
_SC_ROWS = _SC_W * _SC_RPW
_LN2 = 0.6931471805599453
_SQRT2 = 1.4142135623730951


def _sc_ln(x):
    bits = lax.bitcast_convert_type(x, jnp.int32)
    eb = (bits >> 23) & jnp.int32(0xFF)
    mb = (bits & jnp.int32(0x7FFFFF)) | jnp.int32(0x3F800000)
    m = lax.bitcast_convert_type(mb, jnp.float32)
    big = m >= jnp.float32(_SQRT2)
    m = jnp.where(big, m * jnp.float32(0.5), m)
    eb = jnp.where(big, eb + jnp.int32(1), eb)
    ef = lax.bitcast_convert_type(eb + jnp.int32(0x4B000000),
                                  jnp.float32) - jnp.float32(8388608.0 + 127.0)
    d = m + jnp.float32(1.0)
    q = lax.bitcast_convert_type(
        jnp.int32(0x7EF127EA) - lax.bitcast_convert_type(d, jnp.int32),
        jnp.float32)
    q = q * (jnp.float32(2.0) - d * q)
    q = q * (jnp.float32(2.0) - d * q)
    q = q * (jnp.float32(2.0) - d * q)
    r = (m - jnp.float32(1.0)) * q
    r2 = r * r
    p = jnp.float32(1.0 / 9.0)
    p = p * r2 + jnp.float32(1.0 / 7.0)
    p = p * r2 + jnp.float32(1.0 / 5.0)
    p = p * r2 + jnp.float32(1.0 / 3.0)
    p = p * r2 + jnp.float32(1.0)
    return ef * jnp.float32(_LN2) + (r + r) * p


def _sc_allreduce(x, lane, op):
    for k in (8, 4, 2, 1):
        y = x.at[lane ^ jnp.int32(k)].get(mode="promise_in_bounds")
        x = op(x, y)
    return x


def _sc_body(logits_hbm, maski_hbm, out_hbm, lbuf, mbuf, obuf):
    wid = lax.axis_index("s") * jnp.int32(2) + lax.axis_index("c")
    base = wid * jnp.int32(_SC_RPW)
    lane = lax.iota(jnp.int32, 16)
    lane_u = lax.bitcast_convert_type(lane, jnp.uint32)
    neg = jnp.full((16,), _NEG, jnp.float32)
    ninf = jnp.full((16,), -jnp.inf, jnp.float32)
    zf = jnp.zeros((16,), jnp.float32)
    zi = jnp.zeros((16,), jnp.int32)

    def row_step(r, res_vec):
        row = base + r
        pltpu.sync_copy(logits_hbm.at[row], lbuf)
        pltpu.sync_copy(maski_hbm.at[row], mbuf)

        def p1(c, m_acc):
            sl = pl.ds(c * 16, 16)
            msk = jnp.where(mbuf[sl] != 0, lbuf[sl], neg)
            lbuf[sl] = msk
            return jnp.maximum(m_acc, msk)

        m_acc = lax.fori_loop(0, 256, p1, neg)
        msplat = _sc_allreduce(m_acc, lane, jnp.maximum)
        cbase = lax.convert_element_type(row << 12, jnp.uint32)

        def p2(c, carry):
            s_acc, ym, il, vl = carry
            msk = lbuf[pl.ds(c * 16, 16)]
            s_acc = s_acc + jnp.exp(msk - msplat)
            cnt = jnp.full((16,), cbase + lax.convert_element_type(
                c * 16, jnp.uint32), jnp.uint32) + lane_u
            bits = _threefry_bits(cnt)
            fb = (bits >> jnp.uint32(9)) | jnp.uint32(0x3F800000)
            f = lax.bitcast_convert_type(fb, jnp.float32) - jnp.float32(1.0)
            u = f + jnp.float32(_TINY)
            g = -_sc_ln(-_sc_ln(u))
            y = g + msk
            upd = y > ym
            ym = jnp.where(upd, y, ym)
            il = jnp.where(upd, jnp.full((16,), c, jnp.int32), il)
            vl = jnp.where(upd, msk, vl)
            return s_acc, ym, il, vl

        s_acc, ym, il, vl = lax.fori_loop(0, 256, p2, (zf, ninf, zi, neg))
        srow = _sc_allreduce(s_acc, lane, jnp.add)
        lse = msplat + _sc_ln(srow)
        ymax = _sc_allreduce(ym, lane, jnp.maximum)
        eidx = il * jnp.int32(16) + lane
        cand = ym == ymax
        widx = _sc_allreduce(jnp.where(cand, eidx, jnp.int32(1 << 30)),
                             lane, jnp.minimum)
        val = _sc_allreduce(jnp.where(cand & (eidx == widx), vl, ninf),
                            lane, jnp.maximum)
        res = val - lse
        res_vec = jnp.where(lane == (r & 15), res, res_vec)
        obuf[pl.ds((r // 16) * 16, 16)] = res_vec
        return res_vec

    lax.fori_loop(0, _SC_RPW, row_step, zf)
    pltpu.sync_copy(obuf, out_hbm.at[pl.ds(base, _SC_RPW)])


def _sc_sample(logits, maski):
    mesh = plsc.VectorSubcoreMesh(core_axis_name="c", subcore_axis_name="s")
    return pl.kernel(
        _sc_body,
        mesh=mesh,
        out_type=jax.ShapeDtypeStruct((_SC_ROWS,), jnp.float32),
        scratch_types=[
            pltpu.VMEM((4096,), jnp.float32),
            pltpu.VMEM((4096,), jnp.int32),
            pltpu.VMEM((_SC_RPW,), jnp.float32),
        ],
    )(logits, maski)


def kernel(logits, mask):
    b, v = logits.shape
    assert (v & (v - 1)) == 0, "V must be a power of two"
    vshift = v.bit_length() - 1
    blk_r = 256 if b % 256 == 0 else b

    use_sc = v == 4096 and b % 256 == 0 and b > 2 * _SC_ROWS
    sc_rows = _SC_ROWS if use_sc else 0
    tc_rows = b - sc_rows
    blk_off = sc_rows // blk_r

    rowi = jax.lax.broadcasted_iota(jnp.uint32, (blk_r, v), 0)
    coli = jax.lax.broadcasted_iota(jnp.uint32, (blk_r, v), 1)
    blk_iota = (rowi << vshift) | coli

    out_tc = pl.pallas_call(
        functools.partial(_body, vshift=vshift, blk_off=blk_off),
        grid=(tc_rows // blk_r,),
        in_specs=[
            pl.BlockSpec((blk_r, v), lambda i: (0, 0)),
            pl.BlockSpec((blk_r, v), lambda i: (i + blk_off, 0)),
            pl.BlockSpec((blk_r, v), lambda i: (i + blk_off, 0)),
        ],
        out_specs=pl.BlockSpec((blk_r, 1), lambda i: (i, 0)),
        out_shape=jax.ShapeDtypeStruct((tc_rows, 1), jnp.float32),
    )(blk_iota, logits, mask)
    out_tc = out_tc.reshape(tc_rows)
    if not use_sc:
        return out_tc
    out_sc = _sc_sample(logits, mask[:_SC_ROWS].astype(jnp.int32))
    return jnp.concatenate([out_sc, out_tc])


# speedup vs baseline: 1.4606x; 1.0312x over previous
import functools

import jax
import jax.numpy as jnp
from jax import lax
from jax.experimental import pallas as pl
from jax.experimental.pallas import tpu as pltpu
from jax.experimental.pallas import tpu_sc as plsc

_NEG = -1e9
_TINY = 1.1754943508222875e-38
_KS0 = 0x0
_KS1 = 0x2A
_KS2 = _KS0 ^ _KS1 ^ 0x1BD11BDA
_ROT_A = (13, 15, 26, 6)
_ROT_B = (17, 29, 16, 24)


def _rotl(x, d):
    return (x << jnp.uint32(d)) | (x >> jnp.uint32(32 - d))


def _threefry_rounds(x0, x1, rots):
    for r in rots:
        x0 = x0 + x1
        x1 = _rotl(x1, r)
        x1 = x0 ^ x1
    return x0, x1


def _threefry_bits(cnt_lo):
    ks0 = jnp.uint32(_KS0)
    ks1 = jnp.uint32(_KS1)
    ks2 = jnp.uint32(_KS2)
    x1i = cnt_lo + ks1
    x0 = x1i
    x1 = x0 ^ _rotl(x1i, _ROT_A[0])
    x0, x1 = _threefry_rounds(x0, x1, _ROT_A[1:])
    x0, x1 = x0 + ks1, x1 + ks2 + jnp.uint32(1)
    x0, x1 = _threefry_rounds(x0, x1, _ROT_B)
    x0, x1 = x0 + ks2, x1 + ks0 + jnp.uint32(2)
    x0, x1 = _threefry_rounds(x0, x1, _ROT_A)
    x0, x1 = x0 + ks0, x1 + ks1 + jnp.uint32(3)
    x0, x1 = _threefry_rounds(x0, x1, _ROT_B)
    x0, x1 = x0 + ks1, x1 + ks2 + jnp.uint32(4)
    x0, x1 = _threefry_rounds(x0, x1, _ROT_A)
    x0, x1 = x0 + ks2, x1 + ks0 + jnp.uint32(5)
    return x0 ^ x1


def _body(iota_ref, logits_ref, mask_ref, out_ref, *, vshift, blk_off):
    blk_r, v = logits_ref.shape
    logits = logits_ref[...]
    mask = mask_ref[...]
    masked = jnp.where(mask, logits, jnp.float32(_NEG))

    m = jnp.max(masked, axis=1, keepdims=True)
    s = jnp.sum(jnp.exp(masked - m), axis=1, keepdims=True)
    lse = m + jnp.log(s)

    base = (jnp.uint32(pl.program_id(0)) + jnp.uint32(blk_off)) * jnp.uint32(
        blk_r << vshift)
    cnt = iota_ref[...] + base
    bits = _threefry_bits(cnt)

    fb = (bits >> jnp.uint32(9)) | jnp.uint32(0x3F800000)
    f = jax.lax.bitcast_convert_type(fb, jnp.float32) - jnp.float32(1.0)
    u = f + jnp.float32(_TINY)
    g = -jnp.log(-jnp.log(u))

    y = g + masked
    ymax = jnp.max(y, axis=1, keepdims=True)
    val = jnp.max(jnp.where(y == ymax, masked, jnp.float32(-jnp.inf)),
                  axis=1, keepdims=True)
    out_ref[...] = val - lse


_SC_W = 32
_SC_RPW = 96
_SC_ROWS = _SC_W * _SC_RPW
_LN2 = 0.6931471805599453
_SQRT2 = 1.4142135623730951


def _sc_ln(x):
    bits = lax.bitcast_convert_type(x, jnp.int32)
    eb = (bits >> 23) & jnp.int32(0xFF)
    mb = (bits & jnp.int32(0x7FFFFF)) | jnp.int32(0x3F800000)
    m = lax.bitcast_convert_type(mb, jnp.float32)
    big = m >= jnp.float32(_SQRT2)
    m = jnp.where(big, m * jnp.float32(0.5), m)
    eb = jnp.where(big, eb + jnp.int32(1), eb)
    ef = lax.bitcast_convert_type(eb + jnp.int32(0x4B000000),
                                  jnp.float32) - jnp.float32(8388608.0 + 127.0)
    d = m + jnp.float32(1.0)
    q = lax.bitcast_convert_type(
        jnp.int32(0x7EF127EA) - lax.bitcast_convert_type(d, jnp.int32),
        jnp.float32)
    q = q * (jnp.float32(2.0) - d * q)
    q = q * (jnp.float32(2.0) - d * q)
    q = q * (jnp.float32(2.0) - d * q)
    r = (m - jnp.float32(1.0)) * q
    r2 = r * r
    p = jnp.float32(1.0 / 9.0)
    p = p * r2 + jnp.float32(1.0 / 7.0)
    p = p * r2 + jnp.float32(1.0 / 5.0)
    p = p * r2 + jnp.float32(1.0 / 3.0)
    p = p * r2 + jnp.float32(1.0)
    return ef * jnp.float32(_LN2) + (r + r) * p


def _sc_allreduce(x, lane, op):
    for k in (8, 4, 2, 1):
        y = x.at[lane ^ jnp.int32(k)].get(mode="promise_in_bounds")
        x = op(x, y)
    return x


def _sc_body(logits_hbm, maski_hbm, out_hbm, lbuf, mbuf, obuf):
    wid = lax.axis_index("s") * jnp.int32(2) + lax.axis_index("c")
    base = wid * jnp.int32(_SC_RPW)
    lane = lax.iota(jnp.int32, 16)
    lane_u = lax.bitcast_convert_type(lane, jnp.uint32)
    neg = jnp.full((16,), _NEG, jnp.float32)
    ninf = jnp.full((16,), -jnp.inf, jnp.float32)
    zf = jnp.zeros((16,), jnp.float32)
    zi = jnp.zeros((16,), jnp.int32)

    def row_step(r, res_vec):
        row = base + r
        pltpu.sync_copy(logits_hbm.at[row], lbuf)
        pltpu.sync_copy(maski_hbm.at[row], mbuf)

        def p1(c, m_acc):
            sl = pl.ds(c * 16, 16)
            msk = jnp.where(mbuf[sl] != 0, lbuf[sl], neg)
            lbuf[sl] = msk
            return jnp.maximum(m_acc, msk)

        m_acc = lax.fori_loop(0, 256, p1, neg)
        msplat = _sc_allreduce(m_acc, lane, jnp.maximum)
        cbase = lax.convert_element_type(row << 12, jnp.uint32)

        def p2(c, carry):
            s_acc, ym, il, vl = carry
            msk = lbuf[pl.ds(c * 16, 16)]
            s_acc = s_acc + jnp.exp(msk - msplat)
            cnt = jnp.full((16,), cbase + lax.convert_element_type(
                c * 16, jnp.uint32), jnp.uint32) + lane_u
            bits = _threefry_bits(cnt)
            fb = (bits >> jnp.uint32(9)) | jnp.uint32(0x3F800000)
            f = lax.bitcast_convert_type(fb, jnp.float32) - jnp.float32(1.0)
            u = f + jnp.float32(_TINY)
            g = -_sc_ln(-_sc_ln(u))
            y = g + msk
            upd = y > ym
            ym = jnp.where(upd, y, ym)
            il = jnp.where(upd, jnp.full((16,), c, jnp.int32), il)
            vl = jnp.where(upd, msk, vl)
            return s_acc, ym, il, vl

        s_acc, ym, il, vl = lax.fori_loop(0, 256, p2, (zf, ninf, zi, neg))
        srow = _sc_allreduce(s_acc, lane, jnp.add)
        lse = msplat + _sc_ln(srow)
        ymax = _sc_allreduce(ym, lane, jnp.maximum)
        eidx = il * jnp.int32(16) + lane
        cand = ym == ymax
        widx = _sc_allreduce(jnp.where(cand, eidx, jnp.int32(1 << 30)),
                             lane, jnp.minimum)
        val = _sc_allreduce(jnp.where(cand & (eidx == widx), vl, ninf),
                            lane, jnp.maximum)
        res = val - lse
        res_vec = jnp.where(lane == (r & 15), res, res_vec)
        obuf[pl.ds((r // 16) * 16, 16)] = res_vec
        return res_vec

    lax.fori_loop(0, _SC_RPW, row_step, zf)
    pltpu.sync_copy(obuf, out_hbm.at[pl.ds(base, _SC_RPW)])


def _sc_sample(logits, maski):
    mesh = plsc.VectorSubcoreMesh(core_axis_name="c", subcore_axis_name="s")
    return pl.kernel(
        _sc_body,
        mesh=mesh,
        out_type=jax.ShapeDtypeStruct((_SC_ROWS,), jnp.float32),
        scratch_types=[
            pltpu.VMEM((4096,), jnp.float32),
            pltpu.VMEM((4096,), jnp.int32),
            pltpu.VMEM((_SC_RPW,), jnp.float32),
        ],
    )(logits, maski)


def kernel(logits, mask):
    b, v = logits.shape
    assert (v & (v - 1)) == 0, "V must be a power of two"
    vshift = v.bit_length() - 1
    blk_r = 256 if b % 256 == 0 else b

    use_sc = v == 4096 and b % 256 == 0 and b > 2 * _SC_ROWS
    sc_rows = _SC_ROWS if use_sc else 0
    tc_rows = b - sc_rows
    blk_off = sc_rows // blk_r

    rowi = jax.lax.broadcasted_iota(jnp.uint32, (blk_r, v), 0)
    coli = jax.lax.broadcasted_iota(jnp.uint32, (blk_r, v), 1)
    blk_iota = (rowi << vshift) | coli

    out_tc = pl.pallas_call(
        functools.partial(_body, vshift=vshift, blk_off=blk_off),
        grid=(tc_rows // blk_r,),
        in_specs=[
            pl.BlockSpec((blk_r, v), lambda i: (0, 0)),
            pl.BlockSpec((blk_r, v), lambda i: (i + blk_off, 0)),
            pl.BlockSpec((blk_r, v), lambda i: (i + blk_off, 0)),
        ],
        out_specs=pl.BlockSpec((blk_r, 1), lambda i: (i, 0)),
        out_shape=jax.ShapeDtypeStruct((tc_rows, 1), jnp.float32),
    )(blk_iota, logits, mask)
    out_tc = out_tc.reshape(tc_rows)
    if not use_sc:
        return out_tc
    out_sc = _sc_sample(logits, mask[:_SC_ROWS].astype(jnp.int32))
    return jnp.concatenate([out_sc, out_tc])
